# baseline (speedup 1.0000x reference)
import functools

import jax
import jax.numpy as jnp
from jax import lax
from jax.experimental import pallas as pl
from jax.experimental.pallas import tpu as pltpu
from jax.experimental.pallas import tpu_sc as plsc

B = 16384
D = 128
NC = 2
NS = 16
NW = NC * NS
B_PER_W = B // NW
CH = 64
N_CH = B_PER_W // CH
NBUF = 3
VPR = D // 16


def _sc_gather_body(uidx_hbm, iidx_hbm, ue_gmf, ie_gmf, ue_mlp, ie_mlp, wog_hbm,
                    o_gd, o_um, o_im,
                    uix, iix, wog, bufs, obufs, gsem, wsem):
    wid = lax.axis_index("s") * NC + lax.axis_index("c")
    base = wid * B_PER_W

    pltpu.sync_copy(wog_hbm, wog)
    for c in range(N_CH):
        pltpu.sync_copy(uidx_hbm.at[pl.ds(base + c * CH, CH)], uix.at[c])
        pltpu.sync_copy(iidx_hbm.at[pl.ds(base + c * CH, CH)], iix.at[c])

    def gather(c, s):
        bug, big, bum, bim = bufs[s]
        return [
            pltpu.async_copy(ue_gmf.at[uix.at[c]], bug, gsem),
            pltpu.async_copy(ie_gmf.at[iix.at[c]], big, gsem),
            pltpu.async_copy(ue_mlp.at[uix.at[c]], bum, gsem),
            pltpu.async_copy(ie_mlp.at[iix.at[c]], bim, gsem),
        ]

    lane = lax.iota(jnp.int32, 16)
    lane0 = lane == 0
    perms = [lane ^ (1 << k) for k in range(4)]

    _shuf_dnums = lax.GatherDimensionNumbers(
        offset_dims=(), collapsed_slice_dims=(0,), start_index_map=(0,))

    def _lane_sum(v):
        for p in perms:
            v = v + lax.gather(v, p[:, None], _shuf_dnums, (1,),
                               mode=lax.GatherScatterMode.PROMISE_IN_BOUNDS)
        return v

    def gmf_dot(s):
        bug, big = bufs[s][0], bufs[s][1]
        obuf = obufs[s]
        w = [wog[pl.ds(j * 16, 16)] for j in range(VPR)]

        def row(r, res):
            acc = bug[r, pl.ds(0, 16)] * big[r, pl.ds(0, 16)] * w[0]
            for j in range(1, VPR):
                sl = pl.ds(j * 16, 16)
                acc = acc + bug[r, sl] * big[r, sl] * w[j]
            total = _lane_sum(acc)
            res = jnp.where(lane == lax.rem(r, 16), total, res)

            @pl.when(lax.rem(r, 16) == 15)
            def _():
                obuf[pl.ds(r - 15, 16)] = res

            return res

        lax.fori_loop(0, CH, row, jnp.zeros((16,), jnp.float32))

    def writeback(c, s):
        _, _, bum, bim = bufs[s]
        rows = pl.ds(base + c * CH, CH)
        return [
            pltpu.async_copy(obufs[s], o_gd.at[rows], wsem),
            pltpu.async_copy(bum, o_um.at[rows], wsem),
            pltpu.async_copy(bim, o_im.at[rows], wsem),
        ]

    g = {}
    wb = {}
    for c in range(min(2, N_CH)):
        g[c] = gather(c, c % NBUF)
    for c in range(N_CH):
        s = c % NBUF
        for d in g.pop(c):
            d.wait()
        gmf_dot(s)
        wb[s] = writeback(c, s)
        nc = c + 2
        if nc < N_CH:
            ns = nc % NBUF
            if ns in wb:
                for d in wb.pop(ns):
                    d.wait()
            g[nc] = gather(nc, ns)
    for s in list(wb):
        for d in wb.pop(s):
            d.wait()


_sc_gather = functools.partial(
    pl.kernel,
    mesh=plsc.VectorSubcoreMesh(core_axis_name="c", subcore_axis_name="s"),
    out_type=(jax.ShapeDtypeStruct((B,), jnp.float32),
              jax.ShapeDtypeStruct((B, D), jnp.float32),
              jax.ShapeDtypeStruct((B, D), jnp.float32)),
    scratch_types=[
        pltpu.VMEM((N_CH, CH), jnp.int32),
        pltpu.VMEM((N_CH, CH), jnp.int32),
        pltpu.VMEM((D,), jnp.float32),
        tuple(tuple(pltpu.VMEM((CH, D), jnp.float32) for _ in range(4))
              for _ in range(NBUF)),
        tuple(pltpu.VMEM((CH,), jnp.float32) for _ in range(NBUF)),
        pltpu.SemaphoreType.DMA,
        pltpu.SemaphoreType.DMA,
    ],
)(_sc_gather_body)


TC_BLK = 2048

_CONTRACT_01 = (((0,), (1,)), ((), ()))
_CONTRACT_11 = (((1,), (1,)), ((), ()))


def _tc_body(um_r, im_r, w1_r, b1_r, woh_r, bo_r, out_r):
    w1 = w1_r[...]
    h_t = lax.dot_general(w1[0:D, :], um_r[...], _CONTRACT_01,
                          preferred_element_type=jnp.float32)
    h_t = h_t + lax.dot_general(w1[D:2 * D, :], im_r[...], _CONTRACT_01,
                                preferred_element_type=jnp.float32)
    h_t = jnp.maximum(h_t + b1_r[...], 0.0)
    out = jnp.dot(woh_r[...], h_t, preferred_element_type=jnp.float32)
    out_r[...] = out + bo_r[...]


def _tc_forward(um, im, W1, b1_col, woh_row, bo_11):
    grid = (B // TC_BLK,)
    blk = lambda i: (i, 0)
    lane_blk = lambda i: (0, i)
    whole = lambda i: (0, 0)
    out = pl.pallas_call(
        _tc_body,
        grid=grid,
        in_specs=[
            pl.BlockSpec((TC_BLK, D), blk),
            pl.BlockSpec((TC_BLK, D), blk),
            pl.BlockSpec((2 * D, D), whole),
            pl.BlockSpec((D, 1), whole),
            pl.BlockSpec((1, D), whole),
            pl.BlockSpec((1, 1), whole),
        ],
        out_specs=pl.BlockSpec((1, TC_BLK), lane_blk),
        out_shape=jax.ShapeDtypeStruct((1, B), jnp.float32),
    )(um, im, W1, b1_col, woh_row, bo_11)
    return out


def kernel(user_idx, item_idx, ue_gmf, ie_gmf, ue_mlp, ie_mlp, W1, b1, Wo, bo):
    wo_flat = Wo.reshape(2 * D)
    gd, um, im = _sc_gather(user_idx, item_idx, ue_gmf, ie_gmf, ue_mlp, ie_mlp,
                            wo_flat[:D])
    mlp_part = _tc_forward(um, im, W1, b1.reshape(D, 1),
                           wo_flat[D:].reshape(1, D), bo.reshape(1, 1))
    return (mlp_part.reshape(B) + gd).reshape(B, 1)


# speedup vs baseline: 1.0211x; 1.0211x over previous
import functools

import jax
import jax.numpy as jnp
from jax import lax
from jax.experimental import pallas as pl
from jax.experimental.pallas import tpu as pltpu
from jax.experimental.pallas import tpu_sc as plsc

B = 16384
D = 128
NC = 2
NS = 16
NW = NC * NS
B_PER_W = B // NW
CH = 64
N_CH = B_PER_W // CH
NBUF = 3
VPR = D // 16


def _sc_gather_body(uidx_hbm, iidx_hbm, ue_gmf, ie_gmf, ue_mlp, ie_mlp, wog_hbm,
                    o_gd, o_um, o_im,
                    uix, iix, wog, bufs, obufs, gsem, wsem):
    wid = lax.axis_index("s") * NC + lax.axis_index("c")
    base = wid * B_PER_W

    pltpu.sync_copy(wog_hbm, wog)
    for c in range(N_CH):
        pltpu.sync_copy(uidx_hbm.at[pl.ds(base + c * CH, CH)], uix.at[c])
        pltpu.sync_copy(iidx_hbm.at[pl.ds(base + c * CH, CH)], iix.at[c])

    def gather(c, s):
        bug, big, bum, bim = bufs[s]
        return [
            pltpu.async_copy(ue_gmf.at[uix.at[c]], bug, gsem),
            pltpu.async_copy(ie_gmf.at[iix.at[c]], big, gsem),
            pltpu.async_copy(ue_mlp.at[uix.at[c]], bum, gsem),
            pltpu.async_copy(ie_mlp.at[iix.at[c]], bim, gsem),
        ]

    lane = lax.iota(jnp.int32, 16)
    lane0 = lane == 0
    perms = [lane ^ (1 << k) for k in range(4)]

    _shuf_dnums = lax.GatherDimensionNumbers(
        offset_dims=(), collapsed_slice_dims=(0,), start_index_map=(0,))

    def _lane_sum(v):
        for p in perms:
            v = v + lax.gather(v, p[:, None], _shuf_dnums, (1,),
                               mode=lax.GatherScatterMode.PROMISE_IN_BOUNDS)
        return v

    def gmf_dot(s):
        bug, big = bufs[s][0], bufs[s][1]
        obuf = obufs[s]
        w = [wog[pl.ds(j * 16, 16)] for j in range(VPR)]

        def row(r, res):
            acc = bug[r, pl.ds(0, 16)] * big[r, pl.ds(0, 16)] * w[0]
            for j in range(1, VPR):
                sl = pl.ds(j * 16, 16)
                acc = acc + bug[r, sl] * big[r, sl] * w[j]
            total = _lane_sum(acc)
            res = jnp.where(lane == lax.rem(r, 16), total, res)

            @pl.when(lax.rem(r, 16) == 15)
            def _():
                obuf[pl.ds(r - 15, 16)] = res

            return res

        lax.fori_loop(0, CH, row, jnp.zeros((16,), jnp.float32))

    def writeback(c, s):
        _, _, bum, bim = bufs[s]
        rows = pl.ds(base + c * CH, CH)
        return [
            pltpu.async_copy(obufs[s], o_gd.at[rows], wsem),
            pltpu.async_copy(bum, o_um.at[rows], wsem),
            pltpu.async_copy(bim, o_im.at[rows], wsem),
        ]

    g = {}
    wb = {}
    for c in range(min(2, N_CH)):
        g[c] = gather(c, c % NBUF)
    for c in range(N_CH):
        s = c % NBUF
        for d in g.pop(c):
            d.wait()
        gmf_dot(s)
        wb[s] = writeback(c, s)
        nc = c + 2
        if nc < N_CH:
            ns = nc % NBUF
            if ns in wb:
                for d in wb.pop(ns):
                    d.wait()
            g[nc] = gather(nc, ns)
    for s in list(wb):
        for d in wb.pop(s):
            d.wait()


_sc_gather = functools.partial(
    pl.kernel,
    mesh=plsc.VectorSubcoreMesh(core_axis_name="c", subcore_axis_name="s"),
    out_type=(jax.ShapeDtypeStruct((B,), jnp.float32),
              jax.ShapeDtypeStruct((B, D), jnp.float32),
              jax.ShapeDtypeStruct((B, D), jnp.float32)),
    scratch_types=[
        pltpu.VMEM((N_CH, CH), jnp.int32),
        pltpu.VMEM((N_CH, CH), jnp.int32),
        pltpu.VMEM((D,), jnp.float32),
        tuple(tuple(pltpu.VMEM((CH, D), jnp.float32) for _ in range(4))
              for _ in range(NBUF)),
        tuple(pltpu.VMEM((CH,), jnp.float32) for _ in range(NBUF)),
        pltpu.SemaphoreType.DMA,
        pltpu.SemaphoreType.DMA,
    ],
)(_sc_gather_body)


TC_BLK = 2048

_CONTRACT_01 = (((0,), (1,)), ((), ()))
_CONTRACT_11 = (((1,), (1,)), ((), ()))


def _tc_body(um_r, im_r, gd_r, w1_r, b1_r, wo_r, bo_r, out_r):
    w1 = w1_r[...]
    h_t = lax.dot_general(w1[0:D, :], um_r[...], _CONTRACT_01,
                          preferred_element_type=jnp.float32)
    h_t = h_t + lax.dot_general(w1[D:2 * D, :], im_r[...], _CONTRACT_01,
                                preferred_element_type=jnp.float32)
    h_t = jnp.maximum(h_t + b1_r[...].reshape(D, 1), 0.0)
    woh = wo_r[...][D:2 * D, :].T
    out = jnp.dot(woh, h_t, preferred_element_type=jnp.float32)
    out_r[...] = out + gd_r[...] + bo_r[...].reshape(1, 1)


def _tc_forward(um, im, gd_row, W1, b1, Wo, bo):
    grid = (B // TC_BLK,)
    blk = lambda i: (i, 0)
    lane_blk = lambda i: (0, i)
    whole = lambda i: (0, 0)
    out = pl.pallas_call(
        _tc_body,
        grid=grid,
        in_specs=[
            pl.BlockSpec((TC_BLK, D), blk),
            pl.BlockSpec((TC_BLK, D), blk),
            pl.BlockSpec((1, TC_BLK), lane_blk),
            pl.BlockSpec((2 * D, D), whole),
            pl.BlockSpec((D,), lambda i: (0,)),
            pl.BlockSpec((2 * D, 1), whole),
            pl.BlockSpec((1,), lambda i: (0,)),
        ],
        out_specs=pl.BlockSpec((1, TC_BLK), lane_blk),
        out_shape=jax.ShapeDtypeStruct((1, B), jnp.float32),
    )(um, im, gd_row, W1, b1, Wo, bo)
    return out


def kernel(user_idx, item_idx, ue_gmf, ie_gmf, ue_mlp, ie_mlp, W1, b1, Wo, bo):
    gd, um, im = _sc_gather(user_idx, item_idx, ue_gmf, ie_gmf, ue_mlp, ie_mlp,
                            Wo[:D, 0])
    out = _tc_forward(um, im, gd.reshape(1, B), W1, b1, Wo, bo)
    return out.reshape(B, 1)
